# baseline (speedup 1.0000x reference)
import functools

import jax
import jax.numpy as jnp
import numpy as np
from jax import lax
from jax.experimental import pallas as pl
from jax.experimental.pallas import tpu as pltpu
from jax.experimental.pallas import tpu_sc as plsc

B = 16384
V = 14848
D = 128
NSUM = 29
NC = 2
NS = 16
NW = NC * NS
PER_W = B // NW
CB = 4
NCHUNK = PER_W // CB
NCOL = 32
IPC = CB * NCOL
NWORD = D // 2

_PERM = np.empty(D, dtype=np.int32)
for _g in range(D // 32):
    for _l in range(16):
        _PERM[32 * _g + _l] = 32 * _g + 2 * _l
        _PERM[32 * _g + 16 + _l] = 32 * _g + 2 * _l + 1


def _pooled_sc(xc, emb):
    mesh = plsc.VectorSubcoreMesh(core_axis_name="c", subcore_axis_name="s")

    @functools.partial(
        pl.kernel,
        out_type=jax.ShapeDtypeStruct((B, D), jnp.float32),
        mesh=mesh,
        scratch_types=[
            pltpu.VMEM((IPC,), jnp.int32),
            pltpu.VMEM((IPC,), jnp.int32),
            pltpu.VMEM((IPC,), jnp.int32),
            pltpu.VMEM((3, IPC, D), jnp.float32),
            pltpu.VMEM((3, CB, D), jnp.float32),
        ] + [pltpu.SemaphoreType.DMA] * 9,
    )
    def k(xc_hbm, emb_hbm, out_hbm, idx0, idx1, idx2, rows_v, acc_v,
          sem0, sem1, sem2, isem0, isem1, isem2, osem0, osem1, osem2):
        wid = lax.axis_index("s") * NC + lax.axis_index("c")
        obase = wid * PER_W
        ibase = obase * NCOL
        idxs = (idx0, idx1, idx2)
        sems = (sem0, sem1, sem2)
        isems = (isem0, isem1, isem2)
        osems = (osem0, osem1, osem2)

        for b in range(3):
            pltpu.async_copy(
                xc_hbm.at[pl.ds(ibase + b * IPC, IPC)], idxs[b], isems[b])
        for b in range(3):
            pltpu.make_async_copy(
                xc_hbm.at[pl.ds(ibase, IPC)], idxs[b], isems[b]).wait()
            pltpu.async_copy(emb_hbm.at[idxs[b]], rows_v.at[b], sems[b])

        def do_chunk(c, b):
            pltpu.make_async_copy(
                emb_hbm.at[idxs[b]], rows_v.at[b], sems[b]).wait()

            @pl.when(c + 3 < NCHUNK)
            def _():
                pltpu.async_copy(
                    xc_hbm.at[pl.ds(ibase + (c + 3) * IPC, IPC)],
                    idxs[b], isems[b])

            @pl.when(c >= 3)
            def _():
                pltpu.make_async_copy(
                    acc_v.at[b],
                    out_hbm.at[pl.ds(obase + (c - 3) * CB, CB)],
                    osems[b],
                ).wait()

            def row_body(r, _):
                def col_body(j, acc, r=r):
                    src = r * NCOL + j
                    return tuple(
                        acc[g] + rows_v[b, src, pl.ds(g * 16, 16)]
                        for g in range(8)
                    )

                acc = lax.fori_loop(
                    0, NSUM, col_body,
                    tuple(jnp.zeros((16,), jnp.float32) for _ in range(8)),
                )
                for g in range(8):
                    acc_v[b, r, pl.ds(g * 16, 16)] = acc[g]
                return _

            lax.fori_loop(0, CB, row_body, 0)

            pltpu.async_copy(
                acc_v.at[b], out_hbm.at[pl.ds(obase + c * CB, CB)],
                osems[b],
            )

            @pl.when(c + 3 < NCHUNK)
            def _():
                pltpu.make_async_copy(
                    xc_hbm.at[pl.ds(ibase, IPC)], idxs[b], isems[b]).wait()
                pltpu.async_copy(
                    emb_hbm.at[idxs[b]], rows_v.at[b], sems[b])

        loopn = (NCHUNK // 3) * 3

        @pl.loop(0, loopn, step=3)
        def _chunks(i):
            for b in range(3):
                do_chunk(i + b, b)

        for c in range(loopn, NCHUNK):
            do_chunk(jnp.int32(c), c % 3)

        for c in range(NCHUNK - 3, NCHUNK):
            pltpu.make_async_copy(
                acc_v.at[c % 3],
                out_hbm.at[pl.ds(obase + c * CB, CB)],
                osems[c % 3],
            ).wait()

    return k(xc, emb)


def _mlp_body(h_ref, w2_ref, b2_ref, w3_ref, b3_ref, w4_ref, o_ref):
    h = jnp.maximum(h_ref[...], 0.0)
    h = lax.dot_general(h, w2_ref[...], (((1,), (1,)), ((), ())),
                        preferred_element_type=jnp.float32) + b2_ref[...]
    h = jnp.concatenate([h, -h], axis=-1)
    h = jnp.maximum(h, 0.0)
    h = lax.dot_general(h, w3_ref[...], (((1,), (1,)), ((), ())),
                        preferred_element_type=jnp.float32) + b3_ref[...]
    h = jnp.concatenate([h, -h], axis=-1)
    h = jnp.maximum(h, 0.0)
    o_ref[...] = lax.dot_general(h, w4_ref[...], (((1,), (1,)), ((), ())),
                                 preferred_element_type=jnp.float32)


def _mlp_tc(pooled, W2p, b2, W3, b3, W4):
    blk = 2048
    grid = (B // blk,)
    return pl.pallas_call(
        _mlp_body,
        grid=grid,
        in_specs=[
            pl.BlockSpec((blk, D), lambda i: (i, 0)),
            pl.BlockSpec((32, D), lambda i: (0, 0)),
            pl.BlockSpec((1, 32), lambda i: (0, 0)),
            pl.BlockSpec((32, 64), lambda i: (0, 0)),
            pl.BlockSpec((1, 32), lambda i: (0, 0)),
            pl.BlockSpec((1, 64), lambda i: (0, 0)),
        ],
        out_specs=pl.BlockSpec((blk, 1), lambda i: (i, 0)),
        out_shape=jax.ShapeDtypeStruct((B, 1), jnp.float32),
    )(pooled, W2p, b2.reshape(1, 32), W3, b3.reshape(1, 32), W4)


def kernel(x, emb, W2, b2, W3, b3, W4):
    xc = x.astype(jnp.int32).reshape(-1)
    pooled = _pooled_sc(xc, emb)
    W2p = W2
    return _mlp_tc(pooled, W2p, b2, W3, b3, W4)


# speedup vs baseline: 1.0608x; 1.0608x over previous
import functools

import jax
import jax.numpy as jnp
import numpy as np
from jax import lax
from jax.experimental import pallas as pl
from jax.experimental.pallas import tpu as pltpu
from jax.experimental.pallas import tpu_sc as plsc

B = 16384
V = 14848
D = 128
NSUM = 29
NC = 2
NS = 16
NW = NC * NS
PER_W = B // NW
CB = 8
NCHUNK = PER_W // CB
NCOL = 32
IPC = CB * NCOL
NWORD = D // 2

_PERM = np.empty(D, dtype=np.int32)
for _g in range(D // 32):
    for _l in range(16):
        _PERM[32 * _g + _l] = 32 * _g + 2 * _l
        _PERM[32 * _g + 16 + _l] = 32 * _g + 2 * _l + 1


def _pooled_sc(xc, emb):
    mesh = plsc.VectorSubcoreMesh(core_axis_name="c", subcore_axis_name="s")

    @functools.partial(
        pl.kernel,
        out_type=jax.ShapeDtypeStruct((B, D), jnp.float32),
        mesh=mesh,
        scratch_types=[
            pltpu.VMEM((IPC,), jnp.int32),
            pltpu.VMEM((IPC,), jnp.int32),
            pltpu.VMEM((IPC,), jnp.int32),
            pltpu.VMEM((3, IPC, D), jnp.float32),
            pltpu.VMEM((3, CB, D), jnp.float32),
        ] + [pltpu.SemaphoreType.DMA] * 9,
    )
    def k(xc_hbm, emb_hbm, out_hbm, idx0, idx1, idx2, rows_v, acc_v,
          sem0, sem1, sem2, isem0, isem1, isem2, osem0, osem1, osem2):
        wid = lax.axis_index("s") * NC + lax.axis_index("c")
        obase = wid * PER_W
        ibase = obase * NCOL
        idxs = (idx0, idx1, idx2)
        sems = (sem0, sem1, sem2)
        isems = (isem0, isem1, isem2)
        osems = (osem0, osem1, osem2)

        for b in range(3):
            pltpu.async_copy(
                xc_hbm.at[pl.ds(ibase + b * IPC, IPC)], idxs[b], isems[b])
        for b in range(3):
            pltpu.make_async_copy(
                xc_hbm.at[pl.ds(ibase, IPC)], idxs[b], isems[b]).wait()
            pltpu.async_copy(emb_hbm.at[idxs[b]], rows_v.at[b], sems[b])

        def do_chunk(c, b):
            pltpu.make_async_copy(
                emb_hbm.at[idxs[b]], rows_v.at[b], sems[b]).wait()

            @pl.when(c + 3 < NCHUNK)
            def _():
                pltpu.async_copy(
                    xc_hbm.at[pl.ds(ibase + (c + 3) * IPC, IPC)],
                    idxs[b], isems[b])

            @pl.when(c >= 3)
            def _():
                pltpu.make_async_copy(
                    acc_v.at[b],
                    out_hbm.at[pl.ds(obase + (c - 3) * CB, CB)],
                    osems[b],
                ).wait()

            def row_body(r, _):
                def col_body(j, acc, r=r):
                    src = r * NCOL + j
                    return tuple(
                        acc[g] + rows_v[b, src, pl.ds(g * 16, 16)]
                        for g in range(8)
                    )

                acc = lax.fori_loop(
                    0, NSUM, col_body,
                    tuple(jnp.zeros((16,), jnp.float32) for _ in range(8)),
                )
                for g in range(8):
                    acc_v[b, r, pl.ds(g * 16, 16)] = acc[g]
                return _

            lax.fori_loop(0, CB, row_body, 0)

            pltpu.async_copy(
                acc_v.at[b], out_hbm.at[pl.ds(obase + c * CB, CB)],
                osems[b],
            )

            @pl.when(c + 3 < NCHUNK)
            def _():
                pltpu.make_async_copy(
                    xc_hbm.at[pl.ds(ibase, IPC)], idxs[b], isems[b]).wait()
                pltpu.async_copy(
                    emb_hbm.at[idxs[b]], rows_v.at[b], sems[b])

        loopn = (NCHUNK // 3) * 3

        @pl.loop(0, loopn, step=3)
        def _chunks(i):
            for b in range(3):
                do_chunk(i + b, b)

        for c in range(loopn, NCHUNK):
            do_chunk(jnp.int32(c), c % 3)

        for c in range(NCHUNK - 3, NCHUNK):
            pltpu.make_async_copy(
                acc_v.at[c % 3],
                out_hbm.at[pl.ds(obase + c * CB, CB)],
                osems[c % 3],
            ).wait()

    return k(xc, emb)


def _mlp_body(h_ref, w2_ref, b2_ref, w3_ref, b3_ref, w4_ref, o_ref):
    h = jnp.maximum(h_ref[...], 0.0)
    h = lax.dot_general(h, w2_ref[...], (((1,), (1,)), ((), ())),
                        preferred_element_type=jnp.float32) + b2_ref[...]
    h = jnp.concatenate([h, -h], axis=-1)
    h = jnp.maximum(h, 0.0)
    h = lax.dot_general(h, w3_ref[...], (((1,), (1,)), ((), ())),
                        preferred_element_type=jnp.float32) + b3_ref[...]
    h = jnp.concatenate([h, -h], axis=-1)
    h = jnp.maximum(h, 0.0)
    o_ref[...] = lax.dot_general(h, w4_ref[...], (((1,), (1,)), ((), ())),
                                 preferred_element_type=jnp.float32)


def _mlp_tc(pooled, W2p, b2, W3, b3, W4):
    blk = 4096
    grid = (B // blk,)
    return pl.pallas_call(
        _mlp_body,
        grid=grid,
        in_specs=[
            pl.BlockSpec((blk, D), lambda i: (i, 0)),
            pl.BlockSpec((32, D), lambda i: (0, 0)),
            pl.BlockSpec((1, 32), lambda i: (0, 0)),
            pl.BlockSpec((32, 64), lambda i: (0, 0)),
            pl.BlockSpec((1, 32), lambda i: (0, 0)),
            pl.BlockSpec((1, 64), lambda i: (0, 0)),
        ],
        out_specs=pl.BlockSpec((blk, 1), lambda i: (i, 0)),
        out_shape=jax.ShapeDtypeStruct((B, 1), jnp.float32),
    )(pooled, W2p, b2.reshape(1, 32), W3, b3.reshape(1, 32), W4)


def kernel(x, emb, W2, b2, W3, b3, W4):
    xc = x.astype(jnp.int32).reshape(-1)
    pooled = _pooled_sc(xc, emb)
    W2p = W2
    return _mlp_tc(pooled, W2p, b2, W3, b3, W4)


# speedup vs baseline: 1.1320x; 1.0671x over previous
import functools

import jax
import jax.numpy as jnp
import numpy as np
from jax import lax
from jax.experimental import pallas as pl
from jax.experimental.pallas import tpu as pltpu
from jax.experimental.pallas import tpu_sc as plsc

B = 16384
V = 14848
D = 128
NSUM = 29
NC = 2
NS = 16
NW = NC * NS
PER_W = B // NW
CB = 8
NCHUNK = PER_W // CB
NCOL = 32
IPC = CB * NCOL
NWORD = D // 2

_PERM = np.empty(D, dtype=np.int32)
for _g in range(D // 32):
    for _l in range(16):
        _PERM[32 * _g + _l] = 32 * _g + 2 * _l
        _PERM[32 * _g + 16 + _l] = 32 * _g + 2 * _l + 1


def _pooled_sc(xc, emb):
    mesh = plsc.VectorSubcoreMesh(core_axis_name="c", subcore_axis_name="s")

    @functools.partial(
        pl.kernel,
        out_type=jax.ShapeDtypeStruct((B, D), jnp.float32),
        mesh=mesh,
        scratch_types=[
            pltpu.VMEM((IPC,), jnp.int32),
            pltpu.VMEM((IPC,), jnp.int32),
            pltpu.VMEM((IPC,), jnp.int32),
            pltpu.VMEM((3, IPC, D), jnp.float32),
            pltpu.VMEM((3, CB, D), jnp.float32),
        ] + [pltpu.SemaphoreType.DMA] * 9,
    )
    def k(xc_hbm, emb_hbm, out_hbm, idx0, idx1, idx2, rows_v, acc_v,
          sem0, sem1, sem2, isem0, isem1, isem2, osem0, osem1, osem2):
        wid = lax.axis_index("s") * NC + lax.axis_index("c")
        obase = wid * PER_W
        ibase = obase * NCOL
        idxs = (idx0, idx1, idx2)
        sems = (sem0, sem1, sem2)
        isems = (isem0, isem1, isem2)
        osems = (osem0, osem1, osem2)

        for b in range(3):
            pltpu.async_copy(
                xc_hbm.at[pl.ds(ibase + b * IPC, IPC)], idxs[b], isems[b])
        for b in range(3):
            pltpu.make_async_copy(
                xc_hbm.at[pl.ds(ibase, IPC)], idxs[b], isems[b]).wait()
            pltpu.async_copy(emb_hbm.at[idxs[b]], rows_v.at[b], sems[b])

        def do_chunk(c, b):
            pltpu.make_async_copy(
                emb_hbm.at[idxs[b]], rows_v.at[b], sems[b]).wait()

            @pl.when(c + 3 < NCHUNK)
            def _():
                pltpu.async_copy(
                    xc_hbm.at[pl.ds(ibase + (c + 3) * IPC, IPC)],
                    idxs[b], isems[b])

            @pl.when(c >= 3)
            def _():
                pltpu.make_async_copy(
                    acc_v.at[b],
                    out_hbm.at[pl.ds(obase + (c - 3) * CB, CB)],
                    osems[b],
                ).wait()

            def row_body(r, _):
                def col_body(j, acc, r=r):
                    src = r * NCOL + j
                    return tuple(
                        acc[g] + rows_v[b, src, pl.ds(g * 16, 16)]
                        for g in range(8)
                    )

                acc = lax.fori_loop(
                    0, NSUM, col_body,
                    tuple(jnp.zeros((16,), jnp.float32) for _ in range(8)),
                )
                for g in range(8):
                    acc_v[b, r, pl.ds(g * 16, 16)] = acc[g]
                return _

            lax.fori_loop(0, CB, row_body, 0)

            pltpu.async_copy(
                acc_v.at[b], out_hbm.at[pl.ds(obase + c * CB, CB)],
                osems[b],
            )

            @pl.when(c + 3 < NCHUNK)
            def _():
                pltpu.make_async_copy(
                    xc_hbm.at[pl.ds(ibase, IPC)], idxs[b], isems[b]).wait()
                pltpu.async_copy(
                    emb_hbm.at[idxs[b]], rows_v.at[b], sems[b])

        loopn = (NCHUNK // 3) * 3

        @pl.loop(0, loopn, step=3)
        def _chunks(i):
            for b in range(3):
                do_chunk(i + b, b)

        for c in range(loopn, NCHUNK):
            do_chunk(jnp.int32(c), c % 3)

        for c in range(NCHUNK - 3, NCHUNK):
            pltpu.make_async_copy(
                acc_v.at[c % 3],
                out_hbm.at[pl.ds(obase + c * CB, CB)],
                osems[c % 3],
            ).wait()

    return k(xc, emb)


def _mlp_body(h_ref, w2_ref, b2_ref, w3_ref, b3_ref, w4_ref, o_ref):
    h = jnp.maximum(h_ref[...], 0.0)
    h = lax.dot_general(h, w2_ref[...], (((1,), (1,)), ((), ())),
                        preferred_element_type=jnp.float32) + b2_ref[...]
    h = jnp.concatenate([h, -h], axis=-1)
    h = jnp.maximum(h, 0.0)
    h = lax.dot_general(h, w3_ref[...], (((1,), (1,)), ((), ())),
                        preferred_element_type=jnp.float32) + b3_ref[...]
    h = jnp.concatenate([h, -h], axis=-1)
    h = jnp.maximum(h, 0.0)
    o_ref[...] = lax.dot_general(w4_ref[...], h, (((1,), (1,)), ((), ())),
                                 preferred_element_type=jnp.float32)


def _mlp_tc(pooled, W2p, b2, W3, b3, W4):
    blk = 4096
    grid = (B // blk,)
    return pl.pallas_call(
        _mlp_body,
        grid=grid,
        in_specs=[
            pl.BlockSpec((blk, D), lambda i: (i, 0)),
            pl.BlockSpec((32, D), lambda i: (0, 0)),
            pl.BlockSpec((1, 32), lambda i: (0, 0)),
            pl.BlockSpec((32, 64), lambda i: (0, 0)),
            pl.BlockSpec((1, 32), lambda i: (0, 0)),
            pl.BlockSpec((1, 64), lambda i: (0, 0)),
        ],
        out_specs=pl.BlockSpec((1, blk), lambda i: (0, i)),
        out_shape=jax.ShapeDtypeStruct((1, B), jnp.float32),
    )(pooled, W2p, b2.reshape(1, 32), W3, b3.reshape(1, 32), W4).reshape(B, 1)


def kernel(x, emb, W2, b2, W3, b3, W4):
    xc = x.astype(jnp.int32).reshape(-1)
    pooled = _pooled_sc(xc, emb)
    W2p = W2
    return _mlp_tc(pooled, W2p, b2, W3, b3, W4)


# speedup vs baseline: 1.1335x; 1.0014x over previous
import functools

import jax
import jax.numpy as jnp
from jax import lax
from jax.experimental import pallas as pl
from jax.experimental.pallas import tpu as pltpu
from jax.experimental.pallas import tpu_sc as plsc

B = 16384
V = 14848
D = 128
NSUM = 29
NC = 2
NS = 16
NW = NC * NS
PER_W = B // NW
CB = 8
NCHUNK = PER_W // CB
NCOL = 32
IPC = CB * NCOL


def _pooled_sc(xc, emb):
    mesh = plsc.VectorSubcoreMesh(core_axis_name="c", subcore_axis_name="s")

    @functools.partial(
        pl.kernel,
        out_type=jax.ShapeDtypeStruct((B, D), jnp.float32),
        mesh=mesh,
        scratch_types=[
            pltpu.VMEM((IPC,), jnp.int32),
            pltpu.VMEM((IPC,), jnp.int32),
            pltpu.VMEM((IPC,), jnp.int32),
            pltpu.VMEM((3, IPC, D), jnp.float32),
            pltpu.VMEM((3, CB, D), jnp.float32),
        ] + [pltpu.SemaphoreType.DMA] * 9,
    )
    def k(xc_hbm, emb_hbm, out_hbm, idx0, idx1, idx2, rows_v, acc_v,
          sem0, sem1, sem2, isem0, isem1, isem2, osem0, osem1, osem2):
        wid = lax.axis_index("s") * NC + lax.axis_index("c")
        obase = wid * PER_W
        ibase = obase * NCOL
        idxs = (idx0, idx1, idx2)
        sems = (sem0, sem1, sem2)
        isems = (isem0, isem1, isem2)
        osems = (osem0, osem1, osem2)

        for b in range(3):
            pltpu.async_copy(
                xc_hbm.at[pl.ds(ibase + b * IPC, IPC)], idxs[b], isems[b])
        for b in range(3):
            pltpu.make_async_copy(
                xc_hbm.at[pl.ds(ibase, IPC)], idxs[b], isems[b]).wait()
            pltpu.async_copy(emb_hbm.at[idxs[b]], rows_v.at[b], sems[b])

        def do_chunk(c, b):
            pltpu.make_async_copy(
                emb_hbm.at[idxs[b]], rows_v.at[b], sems[b]).wait()

            @pl.when(c + 3 < NCHUNK)
            def _():
                pltpu.async_copy(
                    xc_hbm.at[pl.ds(ibase + (c + 3) * IPC, IPC)],
                    idxs[b], isems[b])

            @pl.when(c >= 3)
            def _():
                pltpu.make_async_copy(
                    acc_v.at[b],
                    out_hbm.at[pl.ds(obase + (c - 3) * CB, CB)],
                    osems[b],
                ).wait()

            def row_body(r, _):
                def col_body(j, acc, r=r):
                    src = r * NCOL + j
                    return tuple(
                        acc[g] + rows_v[b, src, pl.ds(g * 16, 16)]
                        for g in range(8)
                    )

                acc = lax.fori_loop(
                    0, NSUM, col_body,
                    tuple(jnp.zeros((16,), jnp.float32) for _ in range(8)),
                )
                for g in range(8):
                    acc_v[b, r, pl.ds(g * 16, 16)] = acc[g]
                return _

            lax.fori_loop(0, CB, row_body, 0)

            pltpu.async_copy(
                acc_v.at[b], out_hbm.at[pl.ds(obase + c * CB, CB)],
                osems[b],
            )

            @pl.when(c + 3 < NCHUNK)
            def _():
                pltpu.make_async_copy(
                    xc_hbm.at[pl.ds(ibase, IPC)], idxs[b], isems[b]).wait()
                pltpu.async_copy(
                    emb_hbm.at[idxs[b]], rows_v.at[b], sems[b])

        loopn = (NCHUNK // 3) * 3

        @pl.loop(0, loopn, step=3)
        def _chunks(i):
            for b in range(3):
                do_chunk(i + b, b)

        for c in range(loopn, NCHUNK):
            do_chunk(jnp.int32(c), c % 3)

        for c in range(NCHUNK - 3, NCHUNK):
            pltpu.make_async_copy(
                acc_v.at[c % 3],
                out_hbm.at[pl.ds(obase + c * CB, CB)],
                osems[c % 3],
            ).wait()

    return k(xc, emb)


def _mlp_body(h_ref, w2_ref, b2_ref, w3_ref, b3_ref, w4_ref, o_ref):
    h = jnp.maximum(h_ref[...], 0.0)
    h = lax.dot_general(h, w2_ref[...], (((1,), (1,)), ((), ())),
                        preferred_element_type=jnp.float32) + b2_ref[...]
    h = jnp.concatenate([h, -h], axis=-1)
    h = jnp.maximum(h, 0.0)
    h = lax.dot_general(h, w3_ref[...], (((1,), (1,)), ((), ())),
                        preferred_element_type=jnp.float32) + b3_ref[...]
    h = jnp.concatenate([h, -h], axis=-1)
    h = jnp.maximum(h, 0.0)
    o_ref[...] = lax.dot_general(w4_ref[...], h, (((1,), (1,)), ((), ())),
                                 preferred_element_type=jnp.float32)


def _mlp_tc(pooled, W2, b2, W3, b3, W4):
    blk = 4096
    grid = (B // blk,)
    return pl.pallas_call(
        _mlp_body,
        grid=grid,
        in_specs=[
            pl.BlockSpec((blk, D), lambda i: (i, 0)),
            pl.BlockSpec((32, D), lambda i: (0, 0)),
            pl.BlockSpec((1, 32), lambda i: (0, 0)),
            pl.BlockSpec((32, 64), lambda i: (0, 0)),
            pl.BlockSpec((1, 32), lambda i: (0, 0)),
            pl.BlockSpec((1, 64), lambda i: (0, 0)),
        ],
        out_specs=pl.BlockSpec((1, blk), lambda i: (0, i)),
        out_shape=jax.ShapeDtypeStruct((1, B), jnp.float32),
    )(pooled, W2, b2.reshape(1, 32), W3, b3.reshape(1, 32), W4).reshape(B, 1)


def kernel(x, emb, W2, b2, W3, b3, W4):
    xc = x.astype(jnp.int32).reshape(-1)
    pooled = _pooled_sc(xc, emb)
    return _mlp_tc(pooled, W2, b2, W3, b3, W4)


# speedup vs baseline: 1.1343x; 1.0007x over previous
import functools

import jax
import jax.numpy as jnp
from jax import lax
from jax.experimental import pallas as pl
from jax.experimental.pallas import tpu as pltpu
from jax.experimental.pallas import tpu_sc as plsc

B = 16384
V = 14848
D = 128
NSUM = 29
NC = 2
NS = 16
NW = NC * NS
PER_W = B // NW
CB = 8
NCHUNK = PER_W // CB
NCOL = 32
IPC = CB * NCOL


def _pooled_sc(xc, emb):
    mesh = plsc.VectorSubcoreMesh(core_axis_name="c", subcore_axis_name="s")

    @functools.partial(
        pl.kernel,
        out_type=jax.ShapeDtypeStruct((B, D), jnp.float32),
        mesh=mesh,
        scratch_types=[
            pltpu.VMEM((IPC,), jnp.int32),
            pltpu.VMEM((IPC,), jnp.int32),
            pltpu.VMEM((IPC,), jnp.int32),
            pltpu.VMEM((3, IPC, D), jnp.float32),
            pltpu.VMEM((3, CB, D), jnp.float32),
        ] + [pltpu.SemaphoreType.DMA] * 9,
    )
    def k(xc_hbm, emb_hbm, out_hbm, idx0, idx1, idx2, rows_v, acc_v,
          sem0, sem1, sem2, isem0, isem1, isem2, osem0, osem1, osem2):
        wid = lax.axis_index("s") * NC + lax.axis_index("c")
        obase = wid * PER_W
        ibase = obase * NCOL
        idxs = (idx0, idx1, idx2)
        sems = (sem0, sem1, sem2)
        isems = (isem0, isem1, isem2)
        osems = (osem0, osem1, osem2)

        for b in range(3):
            pltpu.async_copy(
                xc_hbm.at[pl.ds(ibase + b * IPC, IPC)], idxs[b], isems[b])
        for b in range(3):
            pltpu.make_async_copy(
                xc_hbm.at[pl.ds(ibase, IPC)], idxs[b], isems[b]).wait()
            pltpu.async_copy(emb_hbm.at[idxs[b]], rows_v.at[b], sems[b])

        def do_chunk(c, b):
            pltpu.make_async_copy(
                emb_hbm.at[idxs[b]], rows_v.at[b], sems[b]).wait()

            @pl.when(c + 3 < NCHUNK)
            def _():
                pltpu.async_copy(
                    xc_hbm.at[pl.ds(ibase + (c + 3) * IPC, IPC)],
                    idxs[b], isems[b])

            @pl.when(c >= 3)
            def _():
                pltpu.make_async_copy(
                    acc_v.at[b],
                    out_hbm.at[pl.ds(obase + (c - 3) * CB, CB)],
                    osems[b],
                ).wait()

            def row_body(r, _):
                def col_body(j, acc, r=r):
                    src = r * NCOL + j
                    return tuple(
                        acc[g] + rows_v[b, src, pl.ds(g * 16, 16)]
                        for g in range(8)
                    )

                acc = lax.fori_loop(
                    0, NSUM, col_body,
                    tuple(jnp.zeros((16,), jnp.float32) for _ in range(8)),
                )
                for g in range(8):
                    acc_v[b, r, pl.ds(g * 16, 16)] = acc[g]
                return _

            lax.fori_loop(0, CB, row_body, 0)

            pltpu.async_copy(
                acc_v.at[b], out_hbm.at[pl.ds(obase + c * CB, CB)],
                osems[b],
            )

            @pl.when(c + 3 < NCHUNK)
            def _():
                pltpu.make_async_copy(
                    xc_hbm.at[pl.ds(ibase, IPC)], idxs[b], isems[b]).wait()
                pltpu.async_copy(
                    emb_hbm.at[idxs[b]], rows_v.at[b], sems[b])

        loopn = (NCHUNK // 3) * 3

        @pl.loop(0, loopn, step=3)
        def _chunks(i):
            for b in range(3):
                do_chunk(i + b, b)

        for c in range(loopn, NCHUNK):
            do_chunk(jnp.int32(c), c % 3)

        for c in range(NCHUNK - 3, NCHUNK):
            pltpu.make_async_copy(
                acc_v.at[c % 3],
                out_hbm.at[pl.ds(obase + c * CB, CB)],
                osems[c % 3],
            ).wait()

    return k(xc, emb)


def _mlp_body(h_ref, w2_ref, b2_ref, w3_ref, b3_ref, w4_ref, o_ref):
    h = jnp.maximum(h_ref[...], 0.0)
    h = lax.dot_general(h, w2_ref[...], (((1,), (1,)), ((), ())),
                        preferred_element_type=jnp.float32) + b2_ref[...]
    h = jnp.concatenate([h, -h], axis=-1)
    h = jnp.maximum(h, 0.0)
    h = lax.dot_general(h, w3_ref[...], (((1,), (1,)), ((), ())),
                        preferred_element_type=jnp.float32) + b3_ref[...]
    h = jnp.concatenate([h, -h], axis=-1)
    h = jnp.maximum(h, 0.0)
    o_ref[...] = lax.dot_general(w4_ref[...], h, (((1,), (1,)), ((), ())),
                                 preferred_element_type=jnp.float32)


def _mlp_tc(pooled, W2, b2, W3, b3, W4):
    blk = 8192
    grid = (B // blk,)
    return pl.pallas_call(
        _mlp_body,
        grid=grid,
        in_specs=[
            pl.BlockSpec((blk, D), lambda i: (i, 0)),
            pl.BlockSpec((32, D), lambda i: (0, 0)),
            pl.BlockSpec((1, 32), lambda i: (0, 0)),
            pl.BlockSpec((32, 64), lambda i: (0, 0)),
            pl.BlockSpec((1, 32), lambda i: (0, 0)),
            pl.BlockSpec((1, 64), lambda i: (0, 0)),
        ],
        out_specs=pl.BlockSpec((1, blk), lambda i: (0, i)),
        out_shape=jax.ShapeDtypeStruct((1, B), jnp.float32),
    )(pooled, W2, b2.reshape(1, 32), W3, b3.reshape(1, 32), W4).reshape(B, 1)


def kernel(x, emb, W2, b2, W3, b3, W4):
    xc = x.astype(jnp.int32).reshape(-1)
    pooled = _pooled_sc(xc, emb)
    return _mlp_tc(pooled, W2, b2, W3, b3, W4)
